# baseline (speedup 1.0000x reference)
import functools

import jax
import jax.numpy as jnp
from jax import lax
from jax.experimental import pallas as pl
from jax.experimental.pallas import tpu as pltpu
from jax.experimental.pallas import tpu_sc as plsc

_VOCAB = 1000000
_EMBED = 32
_BATCH = 16384
_SEQ = 50
_B = _BATCH * _SEQ
_NW = 32
_BPW = _B // _NW
_CHUNK = 512
_NCH = _BPW // _CHUNK
_NBUF = 3
_EB = _EMBED // 8
_BB = _BATCH // 128
_GB = _CHUNK // 128


@functools.partial(
    pl.kernel,
    out_type=jax.ShapeDtypeStruct((_SEQ, _EB, _BB, 8, 128), jnp.float32),
    mesh=plsc.VectorSubcoreMesh(core_axis_name="c", subcore_axis_name="s"),
    scratch_types=(
        [pltpu.VMEM((_CHUNK,), jnp.int32) for _ in range(_NBUF)]
        + [pltpu.VMEM((_NBUF, _CHUNK, _EMBED), jnp.float32),
           pltpu.VMEM((_NBUF, _EB, _GB, 8, 128), jnp.float32),
           pltpu.SemaphoreType.DMA((_NBUF,)),
           pltpu.SemaphoreType.DMA((_NBUF,)),
           pltpu.SemaphoreType.DMA((_NBUF,))]
    ),
    compiler_params=pltpu.CompilerParams(
        use_tc_tiling_on_sc=False, needs_layout_passes=False),
)
def _embed_lookup(idx_hbm, table_hbm, out_hbm, i0, i1, i2, gbufs, tbufs,
                  sem_i, sem_g, sem_s):
    idx_bufs = [i0, i1, i2]
    wid = lax.axis_index("s") * 2 + lax.axis_index("c")
    base = wid * _BPW
    blk0 = wid * (_BPW // 128)

    def idx_load(c, b):
        src = idx_hbm.at[pl.ds(base + c * _CHUNK, _CHUNK)]
        return pltpu.make_async_copy(src, idx_bufs[b], sem_i.at[b])

    def gather(b):
        return pltpu.make_async_copy(
            table_hbm.at[idx_bufs[b]], gbufs.at[b], sem_g.at[b])

    def transpose(b):
        lanes = lax.iota(jnp.int32, 16)

        @pl.loop(0, _GB * 8)
        def _t(t):
            g = t // 8
            e8 = t % 8
            base = g * 128
            for eb in range(_EB):
                col = jnp.full((16,), eb * 8, jnp.int32) + e8
                for sub in range(8):
                    row = lanes + base + (sub * 16)
                    val = plsc.load_gather(gbufs.at[b], [row, col])
                    tbufs[b, eb, g, e8, pl.ds(sub * 16, 16)] = val

    def stores(c, b):
        blk = blk0 + c * _GB
        s = blk // _BB
        bb = blk % _BB
        return [
            pltpu.make_async_copy(
                tbufs.at[b, eb], out_hbm.at[s, eb, pl.ds(bb, _GB)],
                sem_s.at[b])
            for eb in range(_EB)
        ]

    for b in range(_NBUF):
        idx_load(b, b).start()
    idx_load(0, 0).wait()
    gather(0).start()
    idx_load(1, 1).wait()
    gather(1).start()

    def step(c, b, do_next2, do_refill, do_store_wait):
        if do_next2:
            b2 = (b + 2) % _NBUF
            idx_load(c + 2, b2).wait()
            gather(b2).start()
        gather(b).wait()
        if do_refill:
            idx_load(c + _NBUF, b).start()
        if do_store_wait:
            for d in stores(c - _NBUF, b):
                d.wait()
        transpose(b)
        for d in stores(c, b):
            d.start()

    for c in range(_NBUF):
        step(c, c, True, True, False)

    @pl.loop(_NBUF, _NCH - _NBUF - 2, step=_NBUF)
    def _main(c0):
        for b in range(_NBUF):
            step(c0 + b, b, True, True, True)

    for c in range(_NCH - _NBUF - 2, _NCH):
        step(c, c % _NBUF, c + 2 < _NCH, c + _NBUF < _NCH, True)

    for c in range(_NCH - _NBUF, _NCH):
        for d in stores(c, c % _NBUF):
            d.wait()


def kernel(inputs, table):
    idx_sm = jnp.swapaxes(inputs, 0, 1).reshape(_B).astype(jnp.int32)
    out5 = _embed_lookup(idx_sm, table)
    return out5.transpose(2, 4, 0, 1, 3).reshape(_BATCH, _SEQ, _EMBED)


# speedup vs baseline: 1.5716x; 1.5716x over previous
import functools

import jax
import jax.numpy as jnp
from jax import lax
from jax.experimental import pallas as pl
from jax.experimental.pallas import tpu as pltpu
from jax.experimental.pallas import tpu_sc as plsc

_VOCAB = 1000000
_EMBED = 32
_BATCH = 16384
_SEQ = 50
_B = _BATCH * _SEQ
_NW = 32
_BPW = _B // _NW
_CHUNK = 512
_NCH = _BPW // _CHUNK
_NBUF = 3
_EB = _EMBED // 8
_BB = _BATCH // 128
_GB = _CHUNK // 128


@functools.partial(
    pl.kernel,
    out_type=jax.ShapeDtypeStruct((_SEQ, _EB, _BB * 1024), jnp.float32),
    mesh=plsc.VectorSubcoreMesh(core_axis_name="c", subcore_axis_name="s"),
    scratch_types=(
        [pltpu.VMEM((_CHUNK,), jnp.int32) for _ in range(_NBUF)]
        + [pltpu.VMEM((_NBUF, _CHUNK, _EMBED), jnp.float32),
           pltpu.VMEM((_NBUF, _EB, _GB * 1024), jnp.float32),
           pltpu.SemaphoreType.DMA((_NBUF,)),
           pltpu.SemaphoreType.DMA((_NBUF,)),
           pltpu.SemaphoreType.DMA((_NBUF,))]
    ),
    compiler_params=pltpu.CompilerParams(
        use_tc_tiling_on_sc=False, needs_layout_passes=False),
)
def _embed_lookup(idx_hbm, table_hbm, out_hbm, i0, i1, i2, gbufs, tbufs,
                  sem_i, sem_g, sem_s):
    idx_bufs = [i0, i1, i2]
    wid = lax.axis_index("s") * 2 + lax.axis_index("c")
    base = wid * _BPW
    blk0 = wid * (_BPW // 128)

    def idx_load(c, b):
        src = idx_hbm.at[pl.ds(base + c * _CHUNK, _CHUNK)]
        return pltpu.make_async_copy(src, idx_bufs[b], sem_i.at[b])

    def gather(b):
        return pltpu.make_async_copy(
            table_hbm.at[idx_bufs[b]], gbufs.at[b], sem_g.at[b])

    lanes = lax.iota(jnp.int32, 16)
    _colv = [(lanes + d) % 32 for d in range(_EMBED)]
    _ebv = [c // 8 for c in _colv]
    _invl = [(c % 8) * 128 + lanes for c in _colv]

    def transpose(b):
        @pl.loop(0, _GB * 8)
        def _t(t):
            g = t // 8
            sub = t % 8
            rowv = lanes + (g * 128 + sub * 16)
            off = g * 1024 + sub * 16
            for d in range(_EMBED):
                val = plsc.load_gather(gbufs.at[b], [rowv, _colv[d]])
                plsc.store_scatter(
                    tbufs.at[b], [_ebv[d], _invl[d] + off], val)

    def stores(c, b):
        blk = blk0 + c * _GB
        s = blk // _BB
        bb = blk % _BB
        return [
            pltpu.make_async_copy(
                tbufs.at[b, eb],
                out_hbm.at[s, eb, pl.ds(bb * 1024, _GB * 1024)],
                sem_s.at[b])
            for eb in range(_EB)
        ]

    for b in range(_NBUF):
        idx_load(b, b).start()
    idx_load(0, 0).wait()
    gather(0).start()
    idx_load(1, 1).wait()
    gather(1).start()

    def step(c, b, do_next2, do_refill, do_store_wait):
        if do_next2:
            b2 = (b + 2) % _NBUF
            idx_load(c + 2, b2).wait()
            gather(b2).start()
        gather(b).wait()
        if do_refill:
            idx_load(c + _NBUF, b).start()
        if do_store_wait:
            for d in stores(c - _NBUF, b):
                d.wait()
        transpose(b)
        for d in stores(c, b):
            d.start()

    for c in range(_NBUF):
        step(c, c, True, True, False)

    @pl.loop(_NBUF, _NCH - _NBUF - 2, step=_NBUF)
    def _main(c0):
        for b in range(_NBUF):
            step(c0 + b, b, True, True, True)

    for c in range(_NCH - _NBUF - 2, _NCH):
        step(c, c % _NBUF, c + 2 < _NCH, c + _NBUF < _NCH, True)

    for c in range(_NCH - _NBUF, _NCH):
        for d in stores(c, c % _NBUF):
            d.wait()


def kernel(inputs, table):
    idx_sm = jnp.swapaxes(inputs, 0, 1).reshape(_B).astype(jnp.int32)
    out5 = _embed_lookup(idx_sm, table).reshape(_SEQ, _EB, _BB, 8, 128)
    return out5.transpose(2, 4, 0, 1, 3).reshape(_BATCH, _SEQ, _EMBED)
